# baseline (speedup 1.0000x reference)
import functools

import jax
import jax.numpy as jnp
from jax import lax
from jax.experimental import pallas as pl
from jax.experimental.pallas import tpu as pltpu
from jax.experimental.pallas import tpu_sc as plsc

N_SP = 50000
N_SPH = 10000
E = 320000
D = 128

NC = 2
NS = 16
NW = NC * NS
NSPLIT = 2
EH = E // NSPLIT
EW = EH // NW
C = 40
NCH = EW // C

_mesh = plsc.VectorSubcoreMesh(core_axis_name="c", subcore_axis_name="s")


def _wid():
    return lax.axis_index("s") * NC + lax.axis_index("c")


def _gather_body(table_hbm, idx_hbm, out_hbm, idx_v, buf_v, gsem, ssem):
    w = _wid()
    base = w * EW
    pltpu.sync_copy(idx_hbm.at[pl.ds(base, EW)], idx_v)

    def g_copy(i, p):
        return pltpu.make_async_copy(
            table_hbm.at[idx_v.at[pl.ds(i * C, C)]], buf_v.at[p], gsem)

    def s_copy(i, p):
        return pltpu.make_async_copy(
            buf_v.at[p], out_hbm.at[pl.ds(base + i * C, C), :], ssem)

    g_copy(0, 0).start()

    def chunk(i, _):
        p = lax.rem(i, 2)
        g_copy(i, p).wait()

        @pl.when(i > 0)
        def _():
            s_copy(i - 1, 1 - p).wait()

        @pl.when(i < NCH - 1)
        def _():
            g_copy(i + 1, 1 - p).start()

        s_copy(i, p).start()
        return 0

    lax.fori_loop(0, NCH, chunk, 0)
    s_copy(NCH - 1, (NCH - 1) % 2).wait()


def _gather(table, idx):
    k = functools.partial(
        pl.kernel,
        out_type=jax.ShapeDtypeStruct((EH, D), jnp.float32),
        mesh=_mesh,
        scratch_types=[
            pltpu.VMEM((EW,), jnp.int32),
            pltpu.VMEM((2, C, D), jnp.float32),
            pltpu.SemaphoreType.DMA,
            pltpu.SemaphoreType.DMA,
        ],
    )(_gather_body)
    return k(table, idx)


def _segsum_body(u_hbm, idx_hbm, zero_hbm, out_hbm, acc_sh, idx_v, buf_v,
                 lsem, asem):
    cid = lax.axis_index("c")
    sid = lax.axis_index("s")
    w = sid * NC + cid
    base = w * EW
    @pl.when(sid == 0)
    def _zero():
        pltpu.sync_copy(zero_hbm, acc_sh)

    pltpu.sync_copy(idx_hbm.at[w], idx_v)
    plsc.subcore_barrier()

    def l_copy(i, p):
        return pltpu.make_async_copy(
            u_hbm.at[pl.ds(base + i * C, C), :], buf_v.at[p], lsem)

    def a_start(i, p):
        pltpu.async_copy(buf_v.at[p], acc_sh.at[idx_v.at[i]], asem, add=True)

    def a_wait(i, p):
        pltpu.make_async_copy(buf_v.at[p], acc_sh.at[idx_v.at[i]], asem).wait()

    l_copy(0, 0).start()

    def chunk(i, _):
        p = lax.rem(i, 2)
        l_copy(i, p).wait()

        @pl.when(i > 0)
        def _():
            a_wait(i - 1, 1 - p)

        @pl.when(i < NCH - 1)
        def _():
            l_copy(i + 1, 1 - p).start()

        a_start(i, p)
        return 0

    lax.fori_loop(0, NCH, chunk, 0)
    a_wait(NCH - 1, (NCH - 1) % 2)
    plsc.subcore_barrier()

    @pl.when(sid == 0)
    def _writeback():
        pltpu.sync_copy(acc_sh, out_hbm.at[cid])


def _segsum(u, idx3, zero):
    k = functools.partial(
        pl.kernel,
        out_type=jax.ShapeDtypeStruct((NC, N_SPH, D), jnp.float32),
        mesh=_mesh,
        scratch_types=[
            pltpu.VMEM_SHARED((N_SPH, D), jnp.float32),
            pltpu.VMEM((NCH, C), jnp.int32),
            pltpu.VMEM((2, C, D), jnp.float32),
            pltpu.SemaphoreType.DMA,
            pltpu.SemaphoreType.DMA,
        ],
    )(_segsum_body)
    return k(u, idx3, zero)


def _ln(h, g, b):
    mean = jnp.mean(h, axis=1, keepdims=True)
    var = jnp.mean((h - mean) ** 2, axis=1, keepdims=True)
    return (h - mean) * lax.rsqrt(var + 1e-5) * g + b


def _edge_pass_body(u_ref, sg_ref, g_ref, we_ref, w2_ref, b1_ref, g1_ref,
                    be1_ref, b2_ref, out_ref):
    pre = jnp.dot(u_ref[...], we_ref[...], preferred_element_type=jnp.float32)
    pre = pre + sg_ref[...] + g_ref[...] + b1_ref[...]
    h = _ln(jnp.maximum(pre, 0.0), g1_ref[...], be1_ref[...])
    out_ref[...] = (
        jnp.dot(h, w2_ref[...], preferred_element_type=jnp.float32) + b2_ref[...]
    )


BT = 2000


def _edge_pass(u, sg, g, we, w2, b1, g1, be1, b2):
    grid = (EH // BT,)
    big = pl.BlockSpec((BT, D), lambda i: (i, 0))
    mat = pl.BlockSpec((D, D), lambda i: (0, 0))
    vec = pl.BlockSpec((1, D), lambda i: (0, 0))
    return pl.pallas_call(
        _edge_pass_body,
        grid=grid,
        in_specs=[big, big, big, mat, mat, vec, vec, vec, vec],
        out_specs=big,
        out_shape=jax.ShapeDtypeStruct((EH, D), jnp.float32),
    )(u, sg, g, we, w2, b1, g1, be1, b2)


def _proj_body(x_ref, ws_ref, wr_ref, out_ref):
    i = pl.program_id(0)
    w = jnp.where(i < N_SP // BT, ws_ref[...], wr_ref[...])
    out_ref[...] = jnp.dot(x_ref[...], w, preferred_element_type=jnp.float32)


def _proj(nodes, ws, wr):
    n = N_SP + N_SPH
    grid = (n // BT,)
    return pl.pallas_call(
        _proj_body,
        grid=grid,
        in_specs=[
            pl.BlockSpec((BT, D), lambda i: (i, 0)),
            pl.BlockSpec((D, D), lambda i: (0, 0)),
            pl.BlockSpec((D, D), lambda i: (0, 0)),
        ],
        out_specs=pl.BlockSpec((BT, D), lambda i: (i, 0)),
        out_shape=jax.ShapeDtypeStruct((n, D), jnp.float32),
    )(nodes, ws, wr)


def _node_body(sph_ref, p0_ref, p1_ref, w1s_ref, w1m_ref, w2_ref, wr_ref,
               b1_ref, g1_ref, be1_ref, b2_ref, sph_out, rp_out):
    messages = p0_ref[0] + p0_ref[1] + p1_ref[0] + p1_ref[1]
    pre = (
        jnp.dot(sph_ref[...], w1s_ref[...], preferred_element_type=jnp.float32)
        + jnp.dot(messages, w1m_ref[...], preferred_element_type=jnp.float32)
        + b1_ref[...]
    )
    h = _ln(jnp.maximum(pre, 0.0), g1_ref[...], be1_ref[...])
    new_sph = jnp.dot(h, w2_ref[...], preferred_element_type=jnp.float32) + b2_ref[...]
    sph_out[...] = new_sph
    rp_out[...] = jnp.dot(new_sph, wr_ref[...], preferred_element_type=jnp.float32)


def _node_mlp(sphere, p0, p1, w1s, w1m, w2, wr, b1, g1, be1, b2):
    return pl.pallas_call(
        _node_body,
        out_shape=[
            jax.ShapeDtypeStruct((N_SPH, D), jnp.float32),
            jax.ShapeDtypeStruct((N_SPH, D), jnp.float32),
        ],
    )(sphere, p0, p1, w1s, w1m, w2, wr, b1, g1, be1, b2)


def kernel(nodes, edges, senders, receivers,
           e_w1, e_b1, e_g, e_beta, e_w2, e_b2,
           n_w1, n_b1, n_g, n_beta, n_w2, n_b2):
    we, ws, wr = e_w1[:D], e_w1[D:2 * D], e_w1[2 * D:]
    n_w1s, n_w1m = n_w1[:D], n_w1[D:]
    b1 = e_b1.reshape(1, D)
    g1 = e_g.reshape(1, D)
    be1 = e_beta.reshape(1, D)
    b2 = e_b2.reshape(1, D)
    nb1 = n_b1.reshape(1, D)
    ng1 = n_g.reshape(1, D)
    nbe1 = n_beta.reshape(1, D)
    nb2 = n_b2.reshape(1, D)

    send_h = [senders[h * EH:(h + 1) * EH] for h in range(NSPLIT)]
    recv_h = [receivers[h * EH:(h + 1) * EH] for h in range(NSPLIT)]
    recv3_h = [r.reshape(NW, NCH, C) for r in recv_h]
    zero = jnp.zeros((N_SPH, D), jnp.float32)

    proj = _proj(nodes, ws, wr)
    rp = proj[N_SP:]
    sg = [_gather(proj, s) for s in send_h]
    first_g = [_gather(proj, r + N_SP) for r in recv_h]

    sphere = nodes[N_SP:]
    u = [edges[h * EH:(h + 1) * EH] for h in range(NSPLIT)]
    for t in range(3):
        g = first_g if t == 0 else [_gather(rp, r) for r in recv_h]
        parts = [None, None]
        for h in range(NSPLIT):
            u[h] = _edge_pass(u[h], sg[h], g[h], we, e_w2, b1, g1, be1, b2)
            parts[h] = _segsum(u[h], recv3_h[h], zero)
        sphere, rp = _node_mlp(sphere, parts[0], parts[1], n_w1s, n_w1m,
                               n_w2, wr, nb1, ng1, nbe1, nb2)
    return sphere


# speedup vs baseline: 1.2430x; 1.2430x over previous
import functools

import jax
import jax.numpy as jnp
from jax import lax
from jax.experimental import pallas as pl
from jax.experimental.pallas import tpu as pltpu
from jax.experimental.pallas import tpu_sc as plsc

N_SP = 50000
N_SPH = 10000
E = 320000
D = 128

NC = 2
NS = 16
NW = NC * NS
C = 80
NCH_H = (62, 63)
EH_H = tuple(n * C * NW for n in NCH_H)
EOFF_H = (0, EH_H[0])

_mesh = plsc.VectorSubcoreMesh(core_axis_name="c", subcore_axis_name="s")


def _wid():
    return lax.axis_index("s") * NC + lax.axis_index("c")


def _make_gather_body(nch):
    ew = nch * C

    def body(table_hbm, idx_hbm, out_hbm, idx_v, buf_v, gsem, ssem):
        w = _wid()
        base = w * ew
        pltpu.sync_copy(idx_hbm.at[pl.ds(base, ew)], idx_v)

        def g_copy(i, p):
            return pltpu.make_async_copy(
                table_hbm.at[idx_v.at[pl.ds(i * C, C)]], buf_v.at[p], gsem)

        def s_copy(i, p):
            return pltpu.make_async_copy(
                buf_v.at[p], out_hbm.at[pl.ds(base + i * C, C), :], ssem)

        g_copy(0, 0).start()

        def chunk(i, _):
            p = lax.rem(i, 2)
            g_copy(i, p).wait()

            @pl.when(i > 0)
            def _():
                s_copy(i - 1, 1 - p).wait()

            @pl.when(i < nch - 1)
            def _():
                g_copy(i + 1, 1 - p).start()

            s_copy(i, p).start()
            return 0

        lax.fori_loop(0, nch, chunk, 0)
        s_copy(nch - 1, (nch - 1) % 2).wait()

    return body


def _gather(table, idx, nch):
    ew = nch * C
    k = functools.partial(
        pl.kernel,
        out_type=jax.ShapeDtypeStruct((ew * NW, D), jnp.float32),
        mesh=_mesh,
        scratch_types=[
            pltpu.VMEM((ew,), jnp.int32),
            pltpu.VMEM((2, C, D), jnp.float32),
            pltpu.SemaphoreType.DMA,
            pltpu.SemaphoreType.DMA,
        ],
    )(_make_gather_body(nch))
    return k(table, idx)


def _make_segsum_body(nch):
    ew = nch * C

    def body(u_hbm, idx_hbm, zero_hbm, out_hbm, acc_sh, idx_v, buf_v,
             lsem, asem):
        cid = lax.axis_index("c")
        sid = lax.axis_index("s")
        w = sid * NC + cid
        base = w * ew
        @pl.when(sid == 0)
        def _zero():
            pltpu.sync_copy(zero_hbm, acc_sh)

        pltpu.sync_copy(idx_hbm.at[w], idx_v)
        plsc.subcore_barrier()

        def l_copy(i, p):
            return pltpu.make_async_copy(
                u_hbm.at[pl.ds(base + i * C, C), :], buf_v.at[p], lsem)

        def a_start(i, p):
            pltpu.async_copy(buf_v.at[p], acc_sh.at[idx_v.at[i]], asem,
                             add=True)

        def a_wait(i, p):
            pltpu.make_async_copy(
                buf_v.at[p], acc_sh.at[idx_v.at[i]], asem).wait()

        l_copy(0, 0).start()

        def chunk(i, _):
            p = lax.rem(i, 2)
            l_copy(i, p).wait()

            @pl.when(i > 0)
            def _():
                a_wait(i - 1, 1 - p)

            @pl.when(i < nch - 1)
            def _():
                l_copy(i + 1, 1 - p).start()

            a_start(i, p)
            return 0

        lax.fori_loop(0, nch, chunk, 0)
        a_wait(nch - 1, (nch - 1) % 2)
        plsc.subcore_barrier()

        @pl.when(sid == 0)
        def _writeback():
            pltpu.sync_copy(acc_sh, out_hbm.at[cid])

    return body


def _segsum(u, idx3, zero, nch):
    k = functools.partial(
        pl.kernel,
        out_type=jax.ShapeDtypeStruct((NC, N_SPH, D), jnp.float32),
        mesh=_mesh,
        scratch_types=[
            pltpu.VMEM_SHARED((N_SPH, D), jnp.float32),
            pltpu.VMEM((nch, C), jnp.int32),
            pltpu.VMEM((2, C, D), jnp.float32),
            pltpu.SemaphoreType.DMA,
            pltpu.SemaphoreType.DMA,
        ],
    )(_make_segsum_body(nch))
    return k(u, idx3, zero)


def _ln(h, g, b):
    mean = jnp.mean(h, axis=1, keepdims=True)
    var = jnp.mean((h - mean) ** 2, axis=1, keepdims=True)
    return (h - mean) * lax.rsqrt(var + 1e-5) * g + b


def _edge_pass_body(u_ref, sg_ref, g_ref, we_ref, w2_ref, b1_ref, g1_ref,
                    be1_ref, b2_ref, out_ref):
    pre = jnp.dot(u_ref[...], we_ref[...], preferred_element_type=jnp.float32)
    pre = pre + sg_ref[...] + g_ref[...] + b1_ref[...]
    h = _ln(jnp.maximum(pre, 0.0), g1_ref[...], be1_ref[...])
    out_ref[...] = (
        jnp.dot(h, w2_ref[...], preferred_element_type=jnp.float32) + b2_ref[...]
    )


BT = 2000


def _edge_pass(u, sg, g, we, w2, b1, g1, be1, b2):
    n = u.shape[0]
    bt = 1984 if n % 1984 == 0 else 2016
    grid = (n // bt,)
    big = pl.BlockSpec((bt, D), lambda i: (i, 0))
    mat = pl.BlockSpec((D, D), lambda i: (0, 0))
    vec = pl.BlockSpec((1, D), lambda i: (0, 0))
    return pl.pallas_call(
        _edge_pass_body,
        grid=grid,
        in_specs=[big, big, big, mat, mat, vec, vec, vec, vec],
        out_specs=big,
        out_shape=jax.ShapeDtypeStruct((n, D), jnp.float32),
    )(u, sg, g, we, w2, b1, g1, be1, b2)


def _proj_body(x_ref, ws_ref, wr_ref, out_ref):
    i = pl.program_id(0)
    w = jnp.where(i < N_SP // BT, ws_ref[...], wr_ref[...])
    out_ref[...] = jnp.dot(x_ref[...], w, preferred_element_type=jnp.float32)


def _proj(nodes, ws, wr):
    n = N_SP + N_SPH
    grid = (n // BT,)
    return pl.pallas_call(
        _proj_body,
        grid=grid,
        in_specs=[
            pl.BlockSpec((BT, D), lambda i: (i, 0)),
            pl.BlockSpec((D, D), lambda i: (0, 0)),
            pl.BlockSpec((D, D), lambda i: (0, 0)),
        ],
        out_specs=pl.BlockSpec((BT, D), lambda i: (i, 0)),
        out_shape=jax.ShapeDtypeStruct((n, D), jnp.float32),
    )(nodes, ws, wr)


def _node_body(sph_ref, p0_ref, p1_ref, w1s_ref, w1m_ref, w2_ref, wr_ref,
               b1_ref, g1_ref, be1_ref, b2_ref, sph_out, rp_out):
    messages = p0_ref[0] + p0_ref[1] + p1_ref[0] + p1_ref[1]
    pre = (
        jnp.dot(sph_ref[...], w1s_ref[...], preferred_element_type=jnp.float32)
        + jnp.dot(messages, w1m_ref[...], preferred_element_type=jnp.float32)
        + b1_ref[...]
    )
    h = _ln(jnp.maximum(pre, 0.0), g1_ref[...], be1_ref[...])
    new_sph = jnp.dot(h, w2_ref[...], preferred_element_type=jnp.float32) + b2_ref[...]
    sph_out[...] = new_sph
    rp_out[...] = jnp.dot(new_sph, wr_ref[...], preferred_element_type=jnp.float32)


def _node_mlp(sphere, p0, p1, w1s, w1m, w2, wr, b1, g1, be1, b2):
    return pl.pallas_call(
        _node_body,
        out_shape=[
            jax.ShapeDtypeStruct((N_SPH, D), jnp.float32),
            jax.ShapeDtypeStruct((N_SPH, D), jnp.float32),
        ],
    )(sphere, p0, p1, w1s, w1m, w2, wr, b1, g1, be1, b2)


def kernel(nodes, edges, senders, receivers,
           e_w1, e_b1, e_g, e_beta, e_w2, e_b2,
           n_w1, n_b1, n_g, n_beta, n_w2, n_b2):
    we, ws, wr = e_w1[:D], e_w1[D:2 * D], e_w1[2 * D:]
    n_w1s, n_w1m = n_w1[:D], n_w1[D:]
    b1 = e_b1.reshape(1, D)
    g1 = e_g.reshape(1, D)
    be1 = e_beta.reshape(1, D)
    b2 = e_b2.reshape(1, D)
    nb1 = n_b1.reshape(1, D)
    ng1 = n_g.reshape(1, D)
    nbe1 = n_beta.reshape(1, D)
    nb2 = n_b2.reshape(1, D)

    send_h = [senders[o:o + n] for o, n in zip(EOFF_H, EH_H)]
    recv_h = [receivers[o:o + n] for o, n in zip(EOFF_H, EH_H)]
    recv3_h = [r.reshape(NW, nch, C) for r, nch in zip(recv_h, NCH_H)]
    zero = jnp.zeros((N_SPH, D), jnp.float32)

    proj = _proj(nodes, ws, wr)
    rp = proj[N_SP:]
    sg = [_gather(proj, s, nch) for s, nch in zip(send_h, NCH_H)]
    first_g = [_gather(proj, r + N_SP, nch) for r, nch in zip(recv_h, NCH_H)]

    sphere = nodes[N_SP:]
    u = [edges[o:o + n] for o, n in zip(EOFF_H, EH_H)]
    for t in range(3):
        g = (first_g if t == 0
             else [_gather(rp, r, nch) for r, nch in zip(recv_h, NCH_H)])
        parts = [None, None]
        for h in range(2):
            u[h] = _edge_pass(u[h], sg[h], g[h], we, e_w2, b1, g1, be1, b2)
            parts[h] = _segsum(u[h], recv3_h[h], zero, NCH_H[h])
        sphere, rp = _node_mlp(sphere, parts[0], parts[1], n_w1s, n_w1m,
                               n_w2, wr, nb1, ng1, nbe1, nb2)
    return sphere


# speedup vs baseline: 1.4066x; 1.1316x over previous
import functools

import jax
import jax.numpy as jnp
from jax import lax
from jax.experimental import pallas as pl
from jax.experimental.pallas import tpu as pltpu
from jax.experimental.pallas import tpu_sc as plsc

N_SP = 50000
N_SPH = 10000
E = 320000
D = 128

NC = 2
NS = 16
NW = NC * NS
C = 80
NCH_H = (62, 63)
EH_H = tuple(n * C * NW for n in NCH_H)
EOFF_H = (0, EH_H[0])

_mesh = plsc.VectorSubcoreMesh(core_axis_name="c", subcore_axis_name="s")


def _wid():
    return lax.axis_index("s") * NC + lax.axis_index("c")


def _make_gather_body(nch):
    ew = nch * C

    def body(table_hbm, idx_hbm, out_hbm, idx_v, buf_v, gsem, ssem):
        w = _wid()
        base = w * ew
        pltpu.sync_copy(idx_hbm.at[pl.ds(base, ew)], idx_v)

        def g_copy(i, p):
            return pltpu.make_async_copy(
                table_hbm.at[idx_v.at[pl.ds(i * C, C)]], buf_v.at[p], gsem)

        def s_copy(i, p):
            return pltpu.make_async_copy(
                buf_v.at[p], out_hbm.at[pl.ds(base + i * C, C), :], ssem)

        g_copy(0, 0).start()

        def chunk(i, _):
            p = lax.rem(i, 2)
            g_copy(i, p).wait()

            @pl.when(i > 0)
            def _():
                s_copy(i - 1, 1 - p).wait()

            @pl.when(i < nch - 1)
            def _():
                g_copy(i + 1, 1 - p).start()

            s_copy(i, p).start()
            return 0

        lax.fori_loop(0, nch, chunk, 0)
        s_copy(nch - 1, (nch - 1) % 2).wait()

    return body


def _gather(table, idx, nch):
    ew = nch * C
    width = table.shape[1]
    k = functools.partial(
        pl.kernel,
        out_type=jax.ShapeDtypeStruct((ew * NW, width), jnp.float32),
        mesh=_mesh,
        scratch_types=[
            pltpu.VMEM((ew,), jnp.int32),
            pltpu.VMEM((2, C, width), jnp.float32),
            pltpu.SemaphoreType.DMA,
            pltpu.SemaphoreType.DMA,
        ],
    )(_make_gather_body(nch))
    return k(table, idx)


def _make_gather_sh_body(nch):
    ew = nch * C

    def body(table_hbm, idx_hbm, out_hbm, tbl_sh, idx_v, buf_v, gsem, ssem):
        sid = lax.axis_index("s")
        w = _wid()
        base = w * ew
        @pl.when(sid == 0)
        def _stage():
            pltpu.sync_copy(table_hbm, tbl_sh)

        pltpu.sync_copy(idx_hbm.at[pl.ds(base, ew)], idx_v)
        plsc.subcore_barrier()

        def g_copy(i, p):
            return pltpu.make_async_copy(
                tbl_sh.at[idx_v.at[pl.ds(i * C, C)]], buf_v.at[p], gsem)

        def s_copy(i, p):
            return pltpu.make_async_copy(
                buf_v.at[p], out_hbm.at[pl.ds(base + i * C, C), :], ssem)

        g_copy(0, 0).start()

        def chunk(i, _):
            p = lax.rem(i, 2)
            g_copy(i, p).wait()

            @pl.when(i > 0)
            def _():
                s_copy(i - 1, 1 - p).wait()

            @pl.when(i < nch - 1)
            def _():
                g_copy(i + 1, 1 - p).start()

            s_copy(i, p).start()
            return 0

        lax.fori_loop(0, nch, chunk, 0)
        s_copy(nch - 1, (nch - 1) % 2).wait()

    return body


def _gather_sh(table, idx, nch):
    ew = nch * C
    k = functools.partial(
        pl.kernel,
        out_type=jax.ShapeDtypeStruct((ew * NW, D), jnp.float32),
        mesh=_mesh,
        scratch_types=[
            pltpu.VMEM_SHARED((N_SPH, D), jnp.float32),
            pltpu.VMEM((ew,), jnp.int32),
            pltpu.VMEM((2, C, D), jnp.float32),
            pltpu.SemaphoreType.DMA,
            pltpu.SemaphoreType.DMA,
        ],
    )(_make_gather_sh_body(nch))
    return k(table, idx)


def _make_segsum_body(nch):
    ew = nch * C

    def body(u_hbm, idx_hbm, zero_hbm, out_hbm, acc_sh, idx_v, buf_v,
             lsem, asem):
        cid = lax.axis_index("c")
        sid = lax.axis_index("s")
        w = sid * NC + cid
        base = w * ew
        @pl.when(sid == 0)
        def _zero():
            pltpu.sync_copy(zero_hbm, acc_sh)

        pltpu.sync_copy(idx_hbm.at[w], idx_v)
        plsc.subcore_barrier()

        def l_copy(i, p):
            return pltpu.make_async_copy(
                u_hbm.at[pl.ds(base + i * C, C), :], buf_v.at[p], lsem)

        def a_start(i, p):
            pltpu.async_copy(buf_v.at[p], acc_sh.at[idx_v.at[i]], asem,
                             add=True)

        def a_wait(i, p):
            pltpu.make_async_copy(
                buf_v.at[p], acc_sh.at[idx_v.at[i]], asem).wait()

        l_copy(0, 0).start()

        def chunk(i, _):
            p = lax.rem(i, 2)
            l_copy(i, p).wait()

            @pl.when(i > 0)
            def _():
                a_wait(i - 1, 1 - p)

            @pl.when(i < nch - 1)
            def _():
                l_copy(i + 1, 1 - p).start()

            a_start(i, p)
            return 0

        lax.fori_loop(0, nch, chunk, 0)
        a_wait(nch - 1, (nch - 1) % 2)
        plsc.subcore_barrier()

        @pl.when(sid == 0)
        def _writeback():
            pltpu.sync_copy(acc_sh, out_hbm.at[cid])

    return body


def _segsum(u, idx3, zero, nch):
    k = functools.partial(
        pl.kernel,
        out_type=jax.ShapeDtypeStruct((NC, N_SPH, D), jnp.float32),
        mesh=_mesh,
        scratch_types=[
            pltpu.VMEM_SHARED((N_SPH, D), jnp.float32),
            pltpu.VMEM((nch, C), jnp.int32),
            pltpu.VMEM((2, C, D), jnp.float32),
            pltpu.SemaphoreType.DMA,
            pltpu.SemaphoreType.DMA,
        ],
    )(_make_segsum_body(nch))
    return k(u, idx3, zero)


def _ln(h, g, b):
    mean = jnp.mean(h, axis=1, keepdims=True)
    var = jnp.mean((h - mean) ** 2, axis=1, keepdims=True)
    return (h - mean) * lax.rsqrt(var + 1e-5) * g + b


def _edge_pass_body(u_ref, sg_ref, g_ref, we_ref, w2_ref, b1_ref, g1_ref,
                    be1_ref, b2_ref, out_ref):
    pre = jnp.dot(u_ref[...], we_ref[...], preferred_element_type=jnp.float32)
    pre = pre + sg_ref[...] + g_ref[...] + b1_ref[...]
    h = _ln(jnp.maximum(pre, 0.0), g1_ref[...], be1_ref[...])
    out_ref[...] = (
        jnp.dot(h, w2_ref[...], preferred_element_type=jnp.float32) + b2_ref[...]
    )


BT = 2000


def _edge_pass(u, sg, g, we, w2, b1, g1, be1, b2):
    n = u.shape[0]
    bt = 1984 if n % 1984 == 0 else 2016
    grid = (n // bt,)
    big = pl.BlockSpec((bt, D), lambda i: (i, 0))
    mat = pl.BlockSpec((D, D), lambda i: (0, 0))
    vec = pl.BlockSpec((1, D), lambda i: (0, 0))
    return pl.pallas_call(
        _edge_pass_body,
        grid=grid,
        in_specs=[big, big, big, mat, mat, vec, vec, vec, vec],
        out_specs=big,
        out_shape=jax.ShapeDtypeStruct((n, D), jnp.float32),
    )(u, sg, g, we, w2, b1, g1, be1, b2)


def _proj_body(x_ref, ws_ref, wr_ref, out_ref):
    i = pl.program_id(0)
    w = jnp.where(i < N_SP // BT, ws_ref[...], wr_ref[...])
    out_ref[...] = jnp.dot(x_ref[...], w, preferred_element_type=jnp.float32)


def _proj(nodes, ws, wr):
    n = N_SP + N_SPH
    grid = (n // BT,)
    return pl.pallas_call(
        _proj_body,
        grid=grid,
        in_specs=[
            pl.BlockSpec((BT, D), lambda i: (i, 0)),
            pl.BlockSpec((D, D), lambda i: (0, 0)),
            pl.BlockSpec((D, D), lambda i: (0, 0)),
        ],
        out_specs=pl.BlockSpec((BT, D), lambda i: (i, 0)),
        out_shape=jax.ShapeDtypeStruct((n, D), jnp.float32),
    )(nodes, ws, wr)


def _node_body(sph_ref, p0_ref, p1_ref, w1s_ref, w1m_ref, w2_ref, wr_ref,
               b1_ref, g1_ref, be1_ref, b2_ref, sph_out, rp_out):
    messages = p0_ref[0] + p0_ref[1] + p1_ref[0] + p1_ref[1]
    pre = (
        jnp.dot(sph_ref[...], w1s_ref[...], preferred_element_type=jnp.float32)
        + jnp.dot(messages, w1m_ref[...], preferred_element_type=jnp.float32)
        + b1_ref[...]
    )
    h = _ln(jnp.maximum(pre, 0.0), g1_ref[...], be1_ref[...])
    new_sph = jnp.dot(h, w2_ref[...], preferred_element_type=jnp.float32) + b2_ref[...]
    sph_out[...] = new_sph
    rp_out[...] = jnp.dot(new_sph, wr_ref[...],
                          preferred_element_type=jnp.float32)


def _node_mlp(sphere, p0, p1, w1s, w1m, w2, wr, b1, g1, be1, b2):
    return pl.pallas_call(
        _node_body,
        out_shape=[
            jax.ShapeDtypeStruct((N_SPH, D), jnp.float32),
            jax.ShapeDtypeStruct((N_SPH, D), jnp.float32),
        ],
    )(sphere, p0, p1, w1s, w1m, w2, wr, b1, g1, be1, b2)


def kernel(nodes, edges, senders, receivers,
           e_w1, e_b1, e_g, e_beta, e_w2, e_b2,
           n_w1, n_b1, n_g, n_beta, n_w2, n_b2):
    we, ws, wr = e_w1[:D], e_w1[D:2 * D], e_w1[2 * D:]
    n_w1s, n_w1m = n_w1[:D], n_w1[D:]
    b1 = e_b1.reshape(1, D)
    g1 = e_g.reshape(1, D)
    be1 = e_beta.reshape(1, D)
    b2 = e_b2.reshape(1, D)
    nb1 = n_b1.reshape(1, D)
    ng1 = n_g.reshape(1, D)
    nbe1 = n_beta.reshape(1, D)
    nb2 = n_b2.reshape(1, D)

    send_h = [senders[o:o + n] for o, n in zip(EOFF_H, EH_H)]
    recv_h = [receivers[o:o + n] for o, n in zip(EOFF_H, EH_H)]
    recv3_h = [r.reshape(NW, nch, C) for r, nch in zip(recv_h, NCH_H)]
    zero = jnp.zeros((N_SPH, D), jnp.float32)

    proj = _proj(nodes, ws, wr)
    rp = proj[N_SP:]
    sg = [_gather(proj, s, nch) for s, nch in zip(send_h, NCH_H)]

    sphere = nodes[N_SP:]
    u = [edges[o:o + n] for o, n in zip(EOFF_H, EH_H)]
    for t in range(3):
        g = [_gather_sh(rp, r, nch) for r, nch in zip(recv_h, NCH_H)]
        parts = [None, None]
        for h in range(2):
            u[h] = _edge_pass(u[h], sg[h], g[h], we, e_w2, b1, g1, be1, b2)
            parts[h] = _segsum(u[h], recv3_h[h], zero, NCH_H[h])
        sphere, rp = _node_mlp(sphere, parts[0], parts[1], n_w1s, n_w1m,
                               n_w2, wr, nb1, ng1, nbe1, nb2)
    return sphere


# speedup vs baseline: 1.4852x; 1.0559x over previous
import functools

import jax
import jax.numpy as jnp
from jax import lax
from jax.experimental import pallas as pl
from jax.experimental.pallas import tpu as pltpu
from jax.experimental.pallas import tpu_sc as plsc

N_SP = 50000
N_SPH = 10000
E = 320000
D = 128

NC = 2
NS = 16
NW = NC * NS
C = 80
NB = 4
K = 2
NCH_H = (62, 63)
EH_H = tuple(n * C * NW for n in NCH_H)
EOFF_H = (0, EH_H[0])

_mesh = plsc.VectorSubcoreMesh(core_axis_name="c", subcore_axis_name="s")


def _wid():
    return lax.axis_index("s") * NC + lax.axis_index("c")


def _make_gather_body(nch):
    ew = nch * C

    def body(table_hbm, idx_hbm, out_hbm, idx_v, buf_v, gsem, ssem):
        w = _wid()
        base = w * ew
        pltpu.sync_copy(idx_hbm.at[pl.ds(base, ew)], idx_v)

        def g_copy(i, p):
            return pltpu.make_async_copy(
                table_hbm.at[idx_v.at[pl.ds(i * C, C)]], buf_v.at[p], gsem)

        def s_copy(i, p):
            return pltpu.make_async_copy(
                buf_v.at[p], out_hbm.at[pl.ds(base + i * C, C), :], ssem)

        for j in range(NB - K):
            g_copy(j, j).start()

        def chunk(i, _):
            p = lax.rem(i, NB)
            g_copy(i, p).wait()

            @pl.when(i >= K)
            def _():
                s_copy(i - K, lax.rem(i - K, NB)).wait()

            @pl.when(i + NB - K < nch)
            def _():
                g_copy(i + NB - K, lax.rem(i + NB - K, NB)).start()

            s_copy(i, p).start()
            return 0

        lax.fori_loop(0, nch, chunk, 0)

        def drain(i, _):
            s_copy(i, lax.rem(i, NB)).wait()
            return 0

        lax.fori_loop(nch - K, nch, drain, 0)

    return body


def _gather(table, idx, nch):
    ew = nch * C
    width = table.shape[1]
    k = functools.partial(
        pl.kernel,
        out_type=jax.ShapeDtypeStruct((ew * NW, width), jnp.float32),
        mesh=_mesh,
        scratch_types=[
            pltpu.VMEM((ew,), jnp.int32),
            pltpu.VMEM((NB, C, width), jnp.float32),
            pltpu.SemaphoreType.DMA,
            pltpu.SemaphoreType.DMA,
        ],
    )(_make_gather_body(nch))
    return k(table, idx)


def _make_gather_sh_body(nch):
    ew = nch * C

    def body(table_hbm, idx_hbm, out_hbm, tbl_sh, idx_v, buf_v, gsem, ssem):
        sid = lax.axis_index("s")
        w = _wid()
        base = w * ew
        @pl.when(sid == 0)
        def _stage():
            pltpu.sync_copy(table_hbm, tbl_sh)

        pltpu.sync_copy(idx_hbm.at[pl.ds(base, ew)], idx_v)
        plsc.subcore_barrier()

        def g_copy(i, p):
            return pltpu.make_async_copy(
                tbl_sh.at[idx_v.at[pl.ds(i * C, C)]], buf_v.at[p], gsem)

        def s_copy(i, p):
            return pltpu.make_async_copy(
                buf_v.at[p], out_hbm.at[pl.ds(base + i * C, C), :], ssem)

        for j in range(NB - K):
            g_copy(j, j).start()

        def chunk(i, _):
            p = lax.rem(i, NB)
            g_copy(i, p).wait()

            @pl.when(i >= K)
            def _():
                s_copy(i - K, lax.rem(i - K, NB)).wait()

            @pl.when(i + NB - K < nch)
            def _():
                g_copy(i + NB - K, lax.rem(i + NB - K, NB)).start()

            s_copy(i, p).start()
            return 0

        lax.fori_loop(0, nch, chunk, 0)

        def drain(i, _):
            s_copy(i, lax.rem(i, NB)).wait()
            return 0

        lax.fori_loop(nch - K, nch, drain, 0)

    return body


def _gather_sh(table, idx, nch):
    ew = nch * C
    k = functools.partial(
        pl.kernel,
        out_type=jax.ShapeDtypeStruct((ew * NW, D), jnp.float32),
        mesh=_mesh,
        scratch_types=[
            pltpu.VMEM_SHARED((N_SPH, D), jnp.float32),
            pltpu.VMEM((ew,), jnp.int32),
            pltpu.VMEM((NB, C, D), jnp.float32),
            pltpu.SemaphoreType.DMA,
            pltpu.SemaphoreType.DMA,
        ],
    )(_make_gather_sh_body(nch))
    return k(table, idx)


def _make_segsum_body(nch):
    ew = nch * C

    def body(u_hbm, idx_hbm, zero_hbm, out_hbm, acc_sh, idx_v, buf_v,
             lsem, asem):
        cid = lax.axis_index("c")
        sid = lax.axis_index("s")
        w = sid * NC + cid
        base = w * ew
        @pl.when(sid == 0)
        def _zero():
            pltpu.sync_copy(zero_hbm, acc_sh)

        pltpu.sync_copy(idx_hbm.at[w], idx_v)
        plsc.subcore_barrier()

        def l_copy(i, p):
            return pltpu.make_async_copy(
                u_hbm.at[pl.ds(base + i * C, C), :], buf_v.at[p], lsem)

        def a_start(i, p):
            pltpu.async_copy(buf_v.at[p], acc_sh.at[idx_v.at[i]], asem,
                             add=True)

        def a_wait(i, p):
            pltpu.make_async_copy(
                buf_v.at[p], acc_sh.at[idx_v.at[i]], asem).wait()

        for j in range(NB - K):
            l_copy(j, j).start()

        def chunk(i, _):
            p = lax.rem(i, NB)
            l_copy(i, p).wait()

            @pl.when(i >= K)
            def _():
                a_wait(i - K, lax.rem(i - K, NB))

            @pl.when(i + NB - K < nch)
            def _():
                l_copy(i + NB - K, lax.rem(i + NB - K, NB)).start()

            a_start(i, p)
            return 0

        lax.fori_loop(0, nch, chunk, 0)

        def drain(i, _):
            a_wait(i, lax.rem(i, NB))
            return 0

        lax.fori_loop(nch - K, nch, drain, 0)
        plsc.subcore_barrier()

        @pl.when(sid == 0)
        def _writeback():
            pltpu.sync_copy(acc_sh, out_hbm.at[cid])

    return body


def _segsum(u, idx3, zero, nch):
    k = functools.partial(
        pl.kernel,
        out_type=jax.ShapeDtypeStruct((NC, N_SPH, D), jnp.float32),
        mesh=_mesh,
        scratch_types=[
            pltpu.VMEM_SHARED((N_SPH, D), jnp.float32),
            pltpu.VMEM((nch, C), jnp.int32),
            pltpu.VMEM((NB, C, D), jnp.float32),
            pltpu.SemaphoreType.DMA,
            pltpu.SemaphoreType.DMA,
        ],
    )(_make_segsum_body(nch))
    return k(u, idx3, zero)


def _ln(h, g, b):
    mean = jnp.mean(h, axis=1, keepdims=True)
    var = jnp.mean((h - mean) ** 2, axis=1, keepdims=True)
    return (h - mean) * lax.rsqrt(var + 1e-5) * g + b


def _edge_pass_body(u_ref, sg_ref, g_ref, we_ref, w2_ref, b1_ref, g1_ref,
                    be1_ref, b2_ref, out_ref):
    pre = jnp.dot(u_ref[...], we_ref[...], preferred_element_type=jnp.float32)
    pre = pre + sg_ref[...] + g_ref[...] + b1_ref[...]
    h = _ln(jnp.maximum(pre, 0.0), g1_ref[...], be1_ref[...])
    out_ref[...] = (
        jnp.dot(h, w2_ref[...], preferred_element_type=jnp.float32) + b2_ref[...]
    )


BT = 2000


def _edge_pass(u, sg, g, we, w2, b1, g1, be1, b2):
    n = u.shape[0]
    bt = 1984 if n % 1984 == 0 else 2016
    grid = (n // bt,)
    big = pl.BlockSpec((bt, D), lambda i: (i, 0))
    mat = pl.BlockSpec((D, D), lambda i: (0, 0))
    vec = pl.BlockSpec((1, D), lambda i: (0, 0))
    return pl.pallas_call(
        _edge_pass_body,
        grid=grid,
        in_specs=[big, big, big, mat, mat, vec, vec, vec, vec],
        out_specs=big,
        out_shape=jax.ShapeDtypeStruct((n, D), jnp.float32),
    )(u, sg, g, we, w2, b1, g1, be1, b2)


def _proj_body(x_ref, ws_ref, wr_ref, out_ref):
    i = pl.program_id(0)
    w = jnp.where(i < N_SP // BT, ws_ref[...], wr_ref[...])
    out_ref[...] = jnp.dot(x_ref[...], w, preferred_element_type=jnp.float32)


def _proj(nodes, ws, wr):
    n = N_SP + N_SPH
    grid = (n // BT,)
    return pl.pallas_call(
        _proj_body,
        grid=grid,
        in_specs=[
            pl.BlockSpec((BT, D), lambda i: (i, 0)),
            pl.BlockSpec((D, D), lambda i: (0, 0)),
            pl.BlockSpec((D, D), lambda i: (0, 0)),
        ],
        out_specs=pl.BlockSpec((BT, D), lambda i: (i, 0)),
        out_shape=jax.ShapeDtypeStruct((n, D), jnp.float32),
    )(nodes, ws, wr)


def _node_body(sph_ref, p0_ref, p1_ref, w1s_ref, w1m_ref, w2_ref, wr_ref,
               b1_ref, g1_ref, be1_ref, b2_ref, sph_out, rp_out):
    messages = p0_ref[0] + p0_ref[1] + p1_ref[0] + p1_ref[1]
    pre = (
        jnp.dot(sph_ref[...], w1s_ref[...], preferred_element_type=jnp.float32)
        + jnp.dot(messages, w1m_ref[...], preferred_element_type=jnp.float32)
        + b1_ref[...]
    )
    h = _ln(jnp.maximum(pre, 0.0), g1_ref[...], be1_ref[...])
    new_sph = jnp.dot(h, w2_ref[...], preferred_element_type=jnp.float32) + b2_ref[...]
    sph_out[...] = new_sph
    rp_out[...] = jnp.dot(new_sph, wr_ref[...],
                          preferred_element_type=jnp.float32)


def _node_mlp(sphere, p0, p1, w1s, w1m, w2, wr, b1, g1, be1, b2):
    return pl.pallas_call(
        _node_body,
        out_shape=[
            jax.ShapeDtypeStruct((N_SPH, D), jnp.float32),
            jax.ShapeDtypeStruct((N_SPH, D), jnp.float32),
        ],
    )(sphere, p0, p1, w1s, w1m, w2, wr, b1, g1, be1, b2)


def kernel(nodes, edges, senders, receivers,
           e_w1, e_b1, e_g, e_beta, e_w2, e_b2,
           n_w1, n_b1, n_g, n_beta, n_w2, n_b2):
    we, ws, wr = e_w1[:D], e_w1[D:2 * D], e_w1[2 * D:]
    n_w1s, n_w1m = n_w1[:D], n_w1[D:]
    b1 = e_b1.reshape(1, D)
    g1 = e_g.reshape(1, D)
    be1 = e_beta.reshape(1, D)
    b2 = e_b2.reshape(1, D)
    nb1 = n_b1.reshape(1, D)
    ng1 = n_g.reshape(1, D)
    nbe1 = n_beta.reshape(1, D)
    nb2 = n_b2.reshape(1, D)

    send_h = [senders[o:o + n] for o, n in zip(EOFF_H, EH_H)]
    recv_h = [receivers[o:o + n] for o, n in zip(EOFF_H, EH_H)]
    recv3_h = [r.reshape(NW, nch, C) for r, nch in zip(recv_h, NCH_H)]
    zero = jnp.zeros((N_SPH, D), jnp.float32)

    proj = _proj(nodes, ws, wr)
    rp = proj[N_SP:]
    sg = [_gather(proj, s, nch) for s, nch in zip(send_h, NCH_H)]

    sphere = nodes[N_SP:]
    u = [edges[o:o + n] for o, n in zip(EOFF_H, EH_H)]
    for t in range(3):
        g = [_gather_sh(rp, r, nch) for r, nch in zip(recv_h, NCH_H)]
        parts = [None, None]
        for h in range(2):
            u[h] = _edge_pass(u[h], sg[h], g[h], we, e_w2, b1, g1, be1, b2)
            parts[h] = _segsum(u[h], recv3_h[h], zero, NCH_H[h])
        sphere, rp = _node_mlp(sphere, parts[0], parts[1], n_w1s, n_w1m,
                               n_w2, wr, nb1, ng1, nbe1, nb2)
    return sphere


# speedup vs baseline: 1.4878x; 1.0017x over previous
import functools

import jax
import jax.numpy as jnp
from jax import lax
from jax.experimental import pallas as pl
from jax.experimental.pallas import tpu as pltpu
from jax.experimental.pallas import tpu_sc as plsc

N_SP = 50000
N_SPH = 10000
E = 320000
D = 128

NC = 2
NS = 16
NW = NC * NS
C = 80
NB = 4
K = 2
NBG = 8
KG = 4
NCH_H = (62, 63)
EH_H = tuple(n * C * NW for n in NCH_H)
EOFF_H = (0, EH_H[0])

_mesh = plsc.VectorSubcoreMesh(core_axis_name="c", subcore_axis_name="s")


def _wid():
    return lax.axis_index("s") * NC + lax.axis_index("c")


def _make_gather_body(nch):
    ew = nch * C

    def body(table_hbm, idx_hbm, out_hbm, idx_v, buf_v, gsem, ssem):
        w = _wid()
        base = w * ew
        pltpu.sync_copy(idx_hbm.at[pl.ds(base, ew)], idx_v)

        def g_copy(i, p):
            return pltpu.make_async_copy(
                table_hbm.at[idx_v.at[pl.ds(i * C, C)]], buf_v.at[p], gsem)

        def s_copy(i, p):
            return pltpu.make_async_copy(
                buf_v.at[p], out_hbm.at[pl.ds(base + i * C, C), :], ssem)

        for j in range(NBG - KG):
            g_copy(j, j).start()

        def chunk(i, _):
            p = lax.rem(i, NBG)
            g_copy(i, p).wait()

            @pl.when(i >= KG)
            def _():
                s_copy(i - KG, lax.rem(i - KG, NBG)).wait()

            @pl.when(i + NBG - KG < nch)
            def _():
                g_copy(i + NBG - KG, lax.rem(i + NBG - KG, NBG)).start()

            s_copy(i, p).start()
            return 0

        lax.fori_loop(0, nch, chunk, 0)

        def drain(i, _):
            s_copy(i, lax.rem(i, NBG)).wait()
            return 0

        lax.fori_loop(nch - KG, nch, drain, 0)

    return body


def _gather(table, idx, nch):
    ew = nch * C
    width = table.shape[1]
    k = functools.partial(
        pl.kernel,
        out_type=jax.ShapeDtypeStruct((ew * NW, width), jnp.float32),
        mesh=_mesh,
        scratch_types=[
            pltpu.VMEM((ew,), jnp.int32),
            pltpu.VMEM((NBG, C, width), jnp.float32),
            pltpu.SemaphoreType.DMA,
            pltpu.SemaphoreType.DMA,
        ],
    )(_make_gather_body(nch))
    return k(table, idx)


def _make_gather_sh_body(nch):
    ew = nch * C

    def body(table_hbm, idx_hbm, out_hbm, tbl_sh, idx_v, buf_v, gsem, ssem):
        sid = lax.axis_index("s")
        w = _wid()
        base = w * ew
        @pl.when(sid == 0)
        def _stage():
            pltpu.sync_copy(table_hbm, tbl_sh)

        pltpu.sync_copy(idx_hbm.at[pl.ds(base, ew)], idx_v)
        plsc.subcore_barrier()

        def g_copy(i, p):
            return pltpu.make_async_copy(
                tbl_sh.at[idx_v.at[pl.ds(i * C, C)]], buf_v.at[p], gsem)

        def s_copy(i, p):
            return pltpu.make_async_copy(
                buf_v.at[p], out_hbm.at[pl.ds(base + i * C, C), :], ssem)

        for j in range(NB - K):
            g_copy(j, j).start()

        def chunk(i, _):
            p = lax.rem(i, NB)
            g_copy(i, p).wait()

            @pl.when(i >= K)
            def _():
                s_copy(i - K, lax.rem(i - K, NB)).wait()

            @pl.when(i + NB - K < nch)
            def _():
                g_copy(i + NB - K, lax.rem(i + NB - K, NB)).start()

            s_copy(i, p).start()
            return 0

        lax.fori_loop(0, nch, chunk, 0)

        def drain(i, _):
            s_copy(i, lax.rem(i, NB)).wait()
            return 0

        lax.fori_loop(nch - K, nch, drain, 0)

    return body


def _gather_sh(table, idx, nch):
    ew = nch * C
    k = functools.partial(
        pl.kernel,
        out_type=jax.ShapeDtypeStruct((ew * NW, D), jnp.float32),
        mesh=_mesh,
        scratch_types=[
            pltpu.VMEM_SHARED((N_SPH, D), jnp.float32),
            pltpu.VMEM((ew,), jnp.int32),
            pltpu.VMEM((NB, C, D), jnp.float32),
            pltpu.SemaphoreType.DMA,
            pltpu.SemaphoreType.DMA,
        ],
    )(_make_gather_sh_body(nch))
    return k(table, idx)


def _make_segsum_body(nch):
    ew = nch * C

    def body(u_hbm, idx_hbm, zero_hbm, out_hbm, acc_sh, idx_v, buf_v,
             lsem, asem):
        cid = lax.axis_index("c")
        sid = lax.axis_index("s")
        w = sid * NC + cid
        base = w * ew
        @pl.when(sid == 0)
        def _zero():
            pltpu.sync_copy(zero_hbm, acc_sh)

        pltpu.sync_copy(idx_hbm.at[w], idx_v)
        plsc.subcore_barrier()

        def l_copy(i, p):
            return pltpu.make_async_copy(
                u_hbm.at[pl.ds(base + i * C, C), :], buf_v.at[p], lsem)

        def a_start(i, p):
            pltpu.async_copy(buf_v.at[p], acc_sh.at[idx_v.at[i]], asem,
                             add=True)

        def a_wait(i, p):
            pltpu.make_async_copy(
                buf_v.at[p], acc_sh.at[idx_v.at[i]], asem).wait()

        for j in range(NB - K):
            l_copy(j, j).start()

        def chunk(i, _):
            p = lax.rem(i, NB)
            l_copy(i, p).wait()

            @pl.when(i >= K)
            def _():
                a_wait(i - K, lax.rem(i - K, NB))

            @pl.when(i + NB - K < nch)
            def _():
                l_copy(i + NB - K, lax.rem(i + NB - K, NB)).start()

            a_start(i, p)
            return 0

        lax.fori_loop(0, nch, chunk, 0)

        def drain(i, _):
            a_wait(i, lax.rem(i, NB))
            return 0

        lax.fori_loop(nch - K, nch, drain, 0)
        plsc.subcore_barrier()

        @pl.when(sid == 0)
        def _writeback():
            pltpu.sync_copy(acc_sh, out_hbm.at[cid])

    return body


def _segsum(u, idx3, zero, nch):
    k = functools.partial(
        pl.kernel,
        out_type=jax.ShapeDtypeStruct((NC, N_SPH, D), jnp.float32),
        mesh=_mesh,
        scratch_types=[
            pltpu.VMEM_SHARED((N_SPH, D), jnp.float32),
            pltpu.VMEM((nch, C), jnp.int32),
            pltpu.VMEM((NB, C, D), jnp.float32),
            pltpu.SemaphoreType.DMA,
            pltpu.SemaphoreType.DMA,
        ],
    )(_make_segsum_body(nch))
    return k(u, idx3, zero)


def _ln(h, g, b):
    mean = jnp.mean(h, axis=1, keepdims=True)
    var = jnp.mean((h - mean) ** 2, axis=1, keepdims=True)
    return (h - mean) * lax.rsqrt(var + 1e-5) * g + b


def _edge_pass_body(u_ref, sg_ref, g_ref, we_ref, w2_ref, b1_ref, g1_ref,
                    be1_ref, b2_ref, out_ref):
    pre = jnp.dot(u_ref[...], we_ref[...], preferred_element_type=jnp.float32)
    pre = pre + sg_ref[...] + g_ref[...] + b1_ref[...]
    h = _ln(jnp.maximum(pre, 0.0), g1_ref[...], be1_ref[...])
    out_ref[...] = (
        jnp.dot(h, w2_ref[...], preferred_element_type=jnp.float32) + b2_ref[...]
    )


BT = 2000


def _edge_pass(u, sg, g, we, w2, b1, g1, be1, b2):
    n = u.shape[0]
    bt = 1984 if n % 1984 == 0 else 2016
    grid = (n // bt,)
    big = pl.BlockSpec((bt, D), lambda i: (i, 0))
    mat = pl.BlockSpec((D, D), lambda i: (0, 0))
    vec = pl.BlockSpec((1, D), lambda i: (0, 0))
    return pl.pallas_call(
        _edge_pass_body,
        grid=grid,
        in_specs=[big, big, big, mat, mat, vec, vec, vec, vec],
        out_specs=big,
        out_shape=jax.ShapeDtypeStruct((n, D), jnp.float32),
    )(u, sg, g, we, w2, b1, g1, be1, b2)


def _proj_body(x_ref, ws_ref, wr_ref, out_ref):
    i = pl.program_id(0)
    w = jnp.where(i < N_SP // BT, ws_ref[...], wr_ref[...])
    out_ref[...] = jnp.dot(x_ref[...], w, preferred_element_type=jnp.float32)


def _proj(nodes, ws, wr):
    n = N_SP + N_SPH
    grid = (n // BT,)
    return pl.pallas_call(
        _proj_body,
        grid=grid,
        in_specs=[
            pl.BlockSpec((BT, D), lambda i: (i, 0)),
            pl.BlockSpec((D, D), lambda i: (0, 0)),
            pl.BlockSpec((D, D), lambda i: (0, 0)),
        ],
        out_specs=pl.BlockSpec((BT, D), lambda i: (i, 0)),
        out_shape=jax.ShapeDtypeStruct((n, D), jnp.float32),
    )(nodes, ws, wr)


def _node_body(sph_ref, p0_ref, p1_ref, w1s_ref, w1m_ref, w2_ref, wr_ref,
               b1_ref, g1_ref, be1_ref, b2_ref, sph_out, rp_out):
    messages = p0_ref[0] + p0_ref[1] + p1_ref[0] + p1_ref[1]
    pre = (
        jnp.dot(sph_ref[...], w1s_ref[...], preferred_element_type=jnp.float32)
        + jnp.dot(messages, w1m_ref[...], preferred_element_type=jnp.float32)
        + b1_ref[...]
    )
    h = _ln(jnp.maximum(pre, 0.0), g1_ref[...], be1_ref[...])
    new_sph = jnp.dot(h, w2_ref[...], preferred_element_type=jnp.float32) + b2_ref[...]
    sph_out[...] = new_sph
    rp_out[...] = jnp.dot(new_sph, wr_ref[...],
                          preferred_element_type=jnp.float32)


def _node_mlp(sphere, p0, p1, w1s, w1m, w2, wr, b1, g1, be1, b2):
    return pl.pallas_call(
        _node_body,
        out_shape=[
            jax.ShapeDtypeStruct((N_SPH, D), jnp.float32),
            jax.ShapeDtypeStruct((N_SPH, D), jnp.float32),
        ],
    )(sphere, p0, p1, w1s, w1m, w2, wr, b1, g1, be1, b2)


def kernel(nodes, edges, senders, receivers,
           e_w1, e_b1, e_g, e_beta, e_w2, e_b2,
           n_w1, n_b1, n_g, n_beta, n_w2, n_b2):
    we, ws, wr = e_w1[:D], e_w1[D:2 * D], e_w1[2 * D:]
    n_w1s, n_w1m = n_w1[:D], n_w1[D:]
    b1 = e_b1.reshape(1, D)
    g1 = e_g.reshape(1, D)
    be1 = e_beta.reshape(1, D)
    b2 = e_b2.reshape(1, D)
    nb1 = n_b1.reshape(1, D)
    ng1 = n_g.reshape(1, D)
    nbe1 = n_beta.reshape(1, D)
    nb2 = n_b2.reshape(1, D)

    send_h = [senders[o:o + n] for o, n in zip(EOFF_H, EH_H)]
    recv_h = [receivers[o:o + n] for o, n in zip(EOFF_H, EH_H)]
    recv3_h = [r.reshape(NW, nch, C) for r, nch in zip(recv_h, NCH_H)]
    zero = jnp.zeros((N_SPH, D), jnp.float32)

    proj = _proj(nodes, ws, wr)
    rp = proj[N_SP:]
    sg = [_gather(proj, s, nch) for s, nch in zip(send_h, NCH_H)]

    sphere = nodes[N_SP:]
    u = [edges[o:o + n] for o, n in zip(EOFF_H, EH_H)]
    for t in range(3):
        g = [_gather_sh(rp, r, nch) for r, nch in zip(recv_h, NCH_H)]
        parts = [None, None]
        for h in range(2):
            u[h] = _edge_pass(u[h], sg[h], g[h], we, e_w2, b1, g1, be1, b2)
            parts[h] = _segsum(u[h], recv3_h[h], zero, NCH_H[h])
        sphere, rp = _node_mlp(sphere, parts[0], parts[1], n_w1s, n_w1m,
                               n_w2, wr, nb1, ng1, nbe1, nb2)
    return sphere
